# baseline (speedup 1.0000x reference)
import functools

import jax
import jax.numpy as jnp
from jax import lax
from jax.experimental import pallas as pl
from jax.experimental.pallas import tpu as pltpu
from jax.experimental.pallas import tpu_sc as plsc

NC = 1
NS = 16
L = 16
NW = NC * NS

EMB_SZ = 64
BATCH = 16384
B_PER_W = BATCH // NW
CHUNK = 128
N_CHUNKS = B_PER_W // CHUNK
HALF = 256
N_PHASES = B_PER_W // HALF
GROUPS_PER_HALF = HALF // L
BIAS_ROWS = 7813


def _mf_kernel(remb2, pemb2, rbias2, pbias2, pid_hbm, rid_hbm,
               out_hbm,
               rid_v, pid_v, ridx2, pidx2, ridx7, pidx7,
               re_buf, pe_buf, out_v, sem_r, sem_p):
    wid = lax.axis_index("s") * NC + lax.axis_index("c")
    base = wid * B_PER_W

    for j in range(N_CHUNKS):
        pltpu.sync_copy(rid_hbm.at[pl.ds(base + j * CHUNK, CHUNK)], rid_v.at[j])
        pltpu.sync_copy(pid_hbm.at[pl.ds(base + j * CHUNK, CHUNK)], pid_v.at[j])

    for j in range(N_CHUNKS):
        for c in range(CHUNK // L):
            sl = pl.ds(c * L, L)
            r = rid_v[j, sl]
            p = pid_v[j, sl]
            ridx2[j, sl] = r >> 1
            pidx2[j, sl] = p >> 1
            ridx7[j, sl] = r >> 7
            pidx7[j, sl] = p >> 7

    def fire(tbl_r, tbl_p, idx_r, idx_p, h):
        cps = []
        for jj in range(2):
            j = 2 * h + jj
            sl = pl.ds(jj * CHUNK, CHUNK)
            cps.append(pltpu.async_copy(
                tbl_r.at[idx_r.at[j]], re_buf.at[sl], sem_r))
            cps.append(pltpu.async_copy(
                tbl_p.at[idx_p.at[j]], pe_buf.at[sl], sem_p))
        return cps

    def load_ids(id_v, grows):
        return plsc.load_gather(id_v, [grows >> 7, grows & 127])

    def dot_half(h):
        def group(g, carry):
            lrows = lax.iota(jnp.int32, L) + g * L
            grows = lrows + h * HALF
            rid16 = load_ids(rid_v, grows)
            pid16 = load_ids(pid_v, grows)
            rcol = (rid16 & 1) << 6
            pcol = (pid16 & 1) << 6
            acc = jnp.zeros((L,), jnp.float32)
            for d in range(EMB_SZ):
                a = plsc.load_gather(re_buf, [lrows, rcol + d])
                b = plsc.load_gather(pe_buf, [lrows, pcol + d])
                acc = acc + a * b
            out_v[pl.ds(h * HALF + g * L, L)] = acc
            return carry

        lax.fori_loop(0, GROUPS_PER_HALF, group, 0)

    def bias_half(h):
        def group(g, carry):
            lrows = lax.iota(jnp.int32, L) + g * L
            grows = lrows + h * HALF
            rid16 = load_ids(rid_v, grows)
            pid16 = load_ids(pid_v, grows)
            rb = plsc.load_gather(re_buf, [lrows, rid16 & 127])
            pb = plsc.load_gather(pe_buf, [lrows, pid16 & 127])
            sl = pl.ds(h * HALF + g * L, L)
            out_v[sl] = out_v[sl] + rb + pb
            return carry

        lax.fori_loop(0, GROUPS_PER_HALF, group, 0)

    for h in range(N_PHASES):
        cps = fire(remb2, pemb2, ridx2, pidx2, h)
        for c in cps:
            c.wait()
        dot_half(h)
    for h in range(N_PHASES):
        cps = fire(rbias2, pbias2, ridx7, pidx7, h)
        for c in cps:
            c.wait()
        bias_half(h)

    pltpu.sync_copy(out_v, out_hbm.at[pl.ds(base, B_PER_W)])


@jax.jit
def _mf(remb2, pemb2, rbias2, pbias2, product_id, reviewer_id):
    mesh = plsc.VectorSubcoreMesh(core_axis_name="c", subcore_axis_name="s",
                                  num_cores=NC)
    return pl.kernel(
        _mf_kernel,
        out_type=jax.ShapeDtypeStruct((BATCH,), jnp.float32),
        mesh=mesh,
        compiler_params=pltpu.CompilerParams(
            use_tc_tiling_on_sc=True, needs_layout_passes=False),
        scratch_types=[
            pltpu.VMEM((N_CHUNKS, CHUNK), jnp.int32),
            pltpu.VMEM((N_CHUNKS, CHUNK), jnp.int32),
            pltpu.VMEM((N_CHUNKS, CHUNK), jnp.int32),
            pltpu.VMEM((N_CHUNKS, CHUNK), jnp.int32),
            pltpu.VMEM((N_CHUNKS, CHUNK), jnp.int32),
            pltpu.VMEM((N_CHUNKS, CHUNK), jnp.int32),
            pltpu.VMEM((HALF, 128), jnp.float32),
            pltpu.VMEM((HALF, 128), jnp.float32),
            pltpu.VMEM((B_PER_W,), jnp.float32),
            pltpu.SemaphoreType.DMA,
            pltpu.SemaphoreType.DMA,
        ],
    )(remb2, pemb2, rbias2, pbias2, product_id, reviewer_id)


def kernel(reviewer_emb, product_emb, reviewer_bias, product_bias, product_id,
           reviewer_id):
    pad = BIAS_ROWS * 128 - reviewer_bias.shape[0]
    rbias2 = jnp.pad(reviewer_bias.reshape(-1), (0, pad)).reshape(BIAS_ROWS, 128)
    pbias2 = jnp.pad(product_bias.reshape(-1), (0, pad)).reshape(BIAS_ROWS, 128)
    return _mf(reviewer_emb.reshape(500000, 128),
               product_emb.reshape(500000, 128),
               rbias2, pbias2,
               product_id.astype(jnp.int32), reviewer_id.astype(jnp.int32))


# speedup vs baseline: 1.0602x; 1.0602x over previous
import functools

import jax
import jax.numpy as jnp
from jax import lax
from jax.experimental import pallas as pl
from jax.experimental.pallas import tpu as pltpu
from jax.experimental.pallas import tpu_sc as plsc

NC = 2
NS = 16
L = 16
NW = NC * NS

EMB_SZ = 64
BATCH = 16384
B_PER_W = BATCH // NW
CHUNK = 128
N_CHUNKS = B_PER_W // CHUNK
N_GROUPS = B_PER_W // L


def _mf_kernel(remb_hbm, pemb_hbm, rbias_hbm, pbias_hbm, pid_hbm, rid_hbm,
               out_hbm,
               rid_v, pid_v, re_v, pe_v, rb_v, pb_v, out_v,
               sem_re, sem_pe, sem_rb, sem_pb):
    wid = lax.axis_index("s") * NC + lax.axis_index("c")
    base = wid * B_PER_W

    for j in range(N_CHUNKS):
        pltpu.sync_copy(rid_hbm.at[pl.ds(base + j * CHUNK, CHUNK)], rid_v.at[j])
        pltpu.sync_copy(pid_hbm.at[pl.ds(base + j * CHUNK, CHUNK)], pid_v.at[j])

    copies = []
    for j in range(N_CHUNKS):
        sl = pl.ds(j * CHUNK, CHUNK)
        copies.append(pltpu.async_copy(remb_hbm.at[rid_v.at[j]], re_v.at[sl], sem_re))
        copies.append(pltpu.async_copy(pemb_hbm.at[pid_v.at[j]], pe_v.at[sl], sem_pe))
        copies.append(pltpu.async_copy(rbias_hbm.at[rid_v.at[j]], rb_v.at[sl], sem_rb))
        copies.append(pltpu.async_copy(pbias_hbm.at[pid_v.at[j]], pb_v.at[sl], sem_pb))
    for c in copies:
        c.wait()

    def group_body(g, carry):
        rows = lax.iota(jnp.int32, L) + g * L
        acc = rb_v[pl.ds(g * L, L)] + pb_v[pl.ds(g * L, L)]
        for d in range(EMB_SZ):
            dv = jnp.full((L,), d, jnp.int32)
            a = plsc.load_gather(re_v, [rows, dv])
            b = plsc.load_gather(pe_v, [rows, dv])
            acc = acc + a * b
        out_v[pl.ds(g * L, L)] = acc
        return carry

    lax.fori_loop(0, N_GROUPS, group_body, 0)

    pltpu.sync_copy(out_v, out_hbm.at[pl.ds(base, B_PER_W)])


@jax.jit
def _mf(reviewer_emb, product_emb, reviewer_bias, product_bias, product_id,
        reviewer_id):
    mesh = plsc.VectorSubcoreMesh(core_axis_name="c", subcore_axis_name="s")
    return pl.kernel(
        _mf_kernel,
        out_type=jax.ShapeDtypeStruct((BATCH,), jnp.float32),
        mesh=mesh,
        compiler_params=pltpu.CompilerParams(
            needs_layout_passes=False, use_tc_tiling_on_sc=False),
        scratch_types=[
            pltpu.VMEM((N_CHUNKS, CHUNK), jnp.int32),
            pltpu.VMEM((N_CHUNKS, CHUNK), jnp.int32),
            pltpu.VMEM((B_PER_W, EMB_SZ), jnp.float32),
            pltpu.VMEM((B_PER_W, EMB_SZ), jnp.float32),
            pltpu.VMEM((B_PER_W,), jnp.float32),
            pltpu.VMEM((B_PER_W,), jnp.float32),
            pltpu.VMEM((B_PER_W,), jnp.float32),
            pltpu.SemaphoreType.DMA,
            pltpu.SemaphoreType.DMA,
            pltpu.SemaphoreType.DMA,
            pltpu.SemaphoreType.DMA,
        ],
    )(reviewer_emb, product_emb, reviewer_bias, product_bias, product_id,
      reviewer_id)


def kernel(reviewer_emb, product_emb, reviewer_bias, product_bias, product_id,
           reviewer_id):
    return _mf(reviewer_emb, product_emb,
               reviewer_bias.reshape(-1), product_bias.reshape(-1),
               product_id.astype(jnp.int32), reviewer_id.astype(jnp.int32))
